# baseline (speedup 1.0000x reference)
import dataclasses
import functools

import jax
import jax.numpy as jnp
from jax import lax
from jax.experimental import pallas as pl
from jax.experimental.pallas import tpu as pltpu
from jax.experimental.pallas import tpu_sc as plsc

N = 10000
NPAD = 10240
E = 320000
F_IN, F_HID, F_OUT = 128, 256, 128
NC, NS, LANES = 2, 16, 16
K = 128
DH = 128

HALF = NPAD // 2
RACC = HALF + 8
RPT = HALF // NS

CH = 157
EPAD_L1 = NS * CH * K
CH_DEG = 79
EPAD_L2 = NC * NS * CH_DEG * K

TRASH = NPAD

_MESH = plsc.VectorSubcoreMesh(
    core_axis_name="c", subcore_axis_name="s", num_cores=NC, num_subcores=NS
)

_SC_PARAMS = dataclasses.replace(
    pltpu.CompilerParams(), needs_layout_passes=False
)


NDEG = CH_DEG * K


@functools.partial(
    pl.kernel,
    out_type=jax.ShapeDtypeStruct((NC * NS, NPAD), jnp.float32),
    mesh=_MESH,
    scratch_types=[
        pltpu.VMEM((NDEG,), jnp.int32),
        pltpu.VMEM((NPAD + LANES,), jnp.float32),
    ],
    compiler_params=_SC_PARAMS,
)
def _deg_kernel(dst_hbm, out_hbm, idx_v, deg_v):
    c = lax.axis_index("c")
    s = lax.axis_index("s")
    wid = c * NS + s

    @pl.loop(0, NPAD // LANES + 1)
    def _(i):
        deg_v[pl.ds(i * LANES, LANES)] = jnp.zeros((LANES,), jnp.float32)

    pltpu.sync_copy(dst_hbm.at[wid, pl.ds(0, NDEG)], idx_v)

    @pl.loop(0, NDEG // LANES, unroll=2)
    def _(g):
        idx16 = idx_v[pl.ds(g * LANES, LANES)]
        cnt, last = plsc.scan_count(idx16)
        plsc.addupdate_scatter(
            deg_v, [idx16], cnt.astype(jnp.float32), mask=last
        )

    pltpu.sync_copy(deg_v.at[pl.ds(0, NPAD)], out_hbm.at[wid])


CHK = CH * K
KA = 64


@functools.partial(
    pl.kernel,
    out_type=jax.ShapeDtypeStruct((NC, HALF, DH), jnp.float32),
    mesh=_MESH,
    scratch_types=[
        pltpu.VMEM((CHK + LANES,), jnp.int32),
        pltpu.VMEM((CHK + LANES,), jnp.int32),
        pltpu.VMEM((4, KA, DH), jnp.float32),
        pltpu.VMEM_SHARED((RACC, DH), jnp.float32),
        pltpu.VMEM((LANES,), jnp.int32),
        pltpu.SemaphoreType.DMA,
        pltpu.SemaphoreType.DMA,
        pltpu.SemaphoreType.DMA,
        pltpu.SemaphoreType.DMA,
        pltpu.SemaphoreType.DMA,
        pltpu.SemaphoreType.DMA,
    ],
    compiler_params=_SC_PARAMS,
)
def _agg(hs_hbm, src_hbm, dst_hbm, meta_hbm, out_hbm,
         src_v, dst_v, rows_v, acc_sh, meta_v,
         gs0, gs1, gs2, gs3, ss0, ss1):
    c = lax.axis_index("c")
    s = lax.axis_index("s")
    wid = c * NS + s

    pltpu.sync_copy(meta_hbm, meta_v)
    meta = meta_v[...]
    nch = meta[0]
    base = pl.multiple_of(meta[1], HALF)

    pltpu.sync_copy(
        hs_hbm.at[c, pl.ds(base + s * RPT, RPT)],
        acc_sh.at[pl.ds(s * RPT, RPT)],
    )
    pltpu.sync_copy(src_hbm.at[wid], src_v.at[pl.ds(0, CHK)])
    pltpu.sync_copy(dst_hbm.at[wid], dst_v.at[pl.ds(0, CHK)])

    def _group(g, w):
        d = dst_v[pl.ds(g * LANES, LANES)]
        sv = src_v[pl.ds(g * LANES, LANES)]
        local = d - base
        ok = (local >= 0) & (local < HALF)
        plsc.store_compressed(src_v.at[pl.ds(w, LANES)], sv, mask=ok)
        plsc.store_compressed(dst_v.at[pl.ds(w, LANES)], local, mask=ok)
        return w + plsc.all_reduce_population_count(ok)[0]

    w = pl.loop(0, nch * (K // LANES), init_carry=jnp.int32(0))(_group)

    trash8 = HALF + (lax.iota(jnp.int32, LANES) & 7)
    zeros16 = jnp.zeros((LANES,), jnp.int32)
    target = ((w + KA - 1) // KA) * KA

    @pl.loop(0, KA // LANES)
    def _(t):
        pos = w + t * LANES

        @pl.when(pos < target)
        def _():
            src_v[pl.ds(pos, LANES)] = zeros16
            dst_v[pl.ds(pos, LANES)] = trash8

    nq = target // KA
    plsc.subcore_barrier()

    gsem = (gs0, gs1, gs2, gs3)
    ssem = (ss0, ss1)

    def _gather_desc(q, r):
        return pltpu.make_async_copy(
            hs_hbm.at[c].at[src_v.at[pl.ds(q * KA, KA)]], rows_v.at[r], gsem[r]
        )

    def _scatter_desc(q, r):
        return pltpu.make_async_copy(
            rows_v.at[r], acc_sh.at[dst_v.at[pl.ds(q * KA, KA)]], ssem[r % 2]
        )

    def _per_buf(q, fn):
        for r in range(4):
            @pl.when(q % 4 == r)
            def _(r=r):
                fn(q, r)

    @pl.when(nq > 0)
    def _():
        for i in range(2):
            @pl.when(i < nq)
            def _(i=i):
                _gather_desc(i, i).start()

        @pl.loop(0, nq)
        def _(q):
            @pl.when(q >= 2)
            def _():
                _per_buf(q - 2, lambda qq, r: _scatter_desc(qq, r).wait())

            @pl.when(q + 2 < nq)
            def _():
                _per_buf(q + 2, lambda qq, r: _gather_desc(qq, r).start())

            _per_buf(q, lambda qq, r: _gather_desc(qq, r).wait())
            _per_buf(q, lambda qq, r: _scatter_desc(qq, r).start(add=True))

        @pl.when(nq >= 2)
        def _():
            _per_buf(nq - 2, lambda qq, r: _scatter_desc(qq, r).wait())

        _per_buf(nq - 1, lambda qq, r: _scatter_desc(qq, r).wait())

    plsc.subcore_barrier()
    pltpu.sync_copy(
        acc_sh.at[pl.ds(s * RPT, RPT)],
        out_hbm.at[c, pl.ds(s * RPT, RPT)],
    )


_BM = 256
_MB = NPAD // _BM


def _tc1_body(degp_ref, x_ref, w_ref, hs_ref, dinv_ref):
    deg = jnp.sum(degp_ref[...], axis=0) + 1.0
    dinv = lax.rsqrt(deg)[:, None]
    x = x_ref[...]
    ha = jnp.dot(x, w_ref[0], preferred_element_type=jnp.float32)
    hb = jnp.dot(x, w_ref[1], preferred_element_type=jnp.float32)
    hs_ref[...] = jnp.concatenate(
        [(dinv * ha)[None], (dinv * hb)[None]], axis=0
    )
    dinv_ref[...] = dinv


def _tc1_call(degp, x_pad, w1):
    w1h = w1.reshape(F_IN, NC, DH).transpose(1, 0, 2)
    return pl.pallas_call(
        _tc1_body,
        grid=(_MB,),
        in_specs=[
            pl.BlockSpec((NC * NS, _BM), lambda i: (0, i)),
            pl.BlockSpec((_BM, F_IN), lambda i: (i, 0)),
            pl.BlockSpec((NC, F_IN, DH), lambda i: (0, 0, 0)),
        ],
        out_specs=[
            pl.BlockSpec((NC, _BM, DH), lambda i: (0, i, 0)),
            pl.BlockSpec((_BM, 1), lambda i: (i, 0)),
        ],
        out_shape=[
            jax.ShapeDtypeStruct((NC, NPAD, DH), jnp.float32),
            jax.ShapeDtypeStruct((NPAD, 1), jnp.float32),
        ],
    )(degp, x_pad, w1h)


def _tc2_body(acc_ref, dinv_ref, b1_ref, w2_ref, hs2_ref):
    dinv = dinv_ref[...]
    x2 = jnp.concatenate([acc_ref[0], acc_ref[1]], axis=1)
    x2 = jnp.maximum(dinv * x2 + b1_ref[...], 0.0)
    h2 = jnp.dot(x2, w2_ref[...], preferred_element_type=jnp.float32)
    hs2_ref[...] = jnp.broadcast_to((dinv * h2)[None], (NC, _BM, F_OUT))


def _tc2_call(acc1, dinv, b1, w2):
    return pl.pallas_call(
        _tc2_body,
        grid=(_MB,),
        in_specs=[
            pl.BlockSpec((NC, _BM, DH), lambda i: (0, i, 0)),
            pl.BlockSpec((_BM, 1), lambda i: (i, 0)),
            pl.BlockSpec((1, F_HID), lambda i: (0, 0)),
            pl.BlockSpec((F_HID, F_OUT), lambda i: (0, 0)),
        ],
        out_specs=pl.BlockSpec((NC, _BM, F_OUT), lambda i: (0, i, 0)),
        out_shape=jax.ShapeDtypeStruct((NC, NPAD, F_OUT), jnp.float32),
    )(acc1, dinv, b1, w2)


def _tc3_body(acc_ref, hs2_ref, dinv_ref, b2_ref, out_ref):
    y = acc_ref[0] + acc_ref[1] - hs2_ref[0]
    out_ref[...] = dinv_ref[...] * y + b2_ref[...]


def _tc3_call(acc2, hs2, dinv, b2):
    return pl.pallas_call(
        _tc3_body,
        grid=(_MB,),
        in_specs=[
            pl.BlockSpec((NC, _BM, F_OUT), lambda i: (0, i, 0)),
            pl.BlockSpec((1, _BM, F_OUT), lambda i: (0, i, 0)),
            pl.BlockSpec((_BM, 1), lambda i: (i, 0)),
            pl.BlockSpec((1, F_OUT), lambda i: (0, 0)),
        ],
        out_specs=pl.BlockSpec((_BM, F_OUT), lambda i: (i, 0)),
        out_shape=jax.ShapeDtypeStruct((NPAD, F_OUT), jnp.float32),
    )(acc2, hs2, dinv, b2)


def _agg_full(hs, src_t, dst_t, nch):
    halves = []
    for h in range(2):
        meta = jnp.array([nch, h * HALF] + [0] * (LANES - 2), jnp.int32)
        halves.append(_agg(hs, src_t, dst_t, meta))
    return jnp.concatenate(halves, axis=1)


def kernel(x, edge_index, W1, b1, W2, b2):
    src = edge_index[0].astype(jnp.int32)
    dst = edge_index[1].astype(jnp.int32)

    def _flat(v, nway, fill):
        per = E // nway
        body = v.reshape(nway, per)
        padb = jnp.full((nway, CHK - per), fill, jnp.int32)
        return jnp.concatenate([body, padb], axis=1)

    src16 = _flat(src, NS, 0)
    dst16 = _flat(dst, NS, TRASH)
    src_l1 = jnp.concatenate([src16, src16])
    dst_l1 = jnp.concatenate([dst16, dst16])
    src_l2 = _flat(src, NC * NS, 0)
    dst_l2 = _flat(dst, NC * NS, TRASH)
    x_pad = jnp.zeros((NPAD, F_IN), jnp.float32).at[:N].set(x)

    degp = _deg_kernel(dst_l2)
    hs1, dinv = _tc1_call(degp, x_pad, W1)
    acc1 = _agg_full(hs1, src_l1, dst_l1, CH)
    hs2 = _tc2_call(acc1, dinv, b1.reshape(1, F_HID), W2)
    acc2 = _agg_full(hs2, src_l2, dst_l2, CH_DEG)
    out = _tc3_call(acc2, hs2, dinv, b2.reshape(1, F_OUT))
    return out[:N]


# speedup vs baseline: 1.0046x; 1.0046x over previous
import dataclasses
import functools

import jax
import jax.numpy as jnp
from jax import lax
from jax.experimental import pallas as pl
from jax.experimental.pallas import tpu as pltpu
from jax.experimental.pallas import tpu_sc as plsc

N = 10000
NPAD = 10240
E = 320000
F_IN, F_HID, F_OUT = 128, 256, 128
NC, NS, LANES = 2, 16, 16
K = 128
DH = 128

HALF = NPAD // 2
RACC = HALF + 8
RPT = HALF // NS

CH = 157
EPAD_L1 = NS * CH * K
CH_DEG = 79
EPAD_L2 = NC * NS * CH_DEG * K

TRASH = NPAD

_MESH = plsc.VectorSubcoreMesh(
    core_axis_name="c", subcore_axis_name="s", num_cores=NC, num_subcores=NS
)

_SC_PARAMS = dataclasses.replace(
    pltpu.CompilerParams(), needs_layout_passes=False
)


@functools.partial(
    pl.kernel,
    out_type=jax.ShapeDtypeStruct((NC * NS, NPAD), jnp.float32),
    mesh=_MESH,
    scratch_types=[
        pltpu.VMEM((CH_DEG, K), jnp.int32),
        pltpu.VMEM((NPAD + LANES,), jnp.float32),
    ],
    compiler_params=_SC_PARAMS,
)
def _deg_kernel(dst_hbm, out_hbm, idx_v, deg_v):
    c = lax.axis_index("c")
    s = lax.axis_index("s")
    wid = c * NS + s

    @pl.loop(0, NPAD // LANES + 1)
    def _(i):
        deg_v[pl.ds(i * LANES, LANES)] = jnp.zeros((LANES,), jnp.float32)

    pltpu.sync_copy(dst_hbm.at[wid], idx_v)

    @pl.loop(0, CH_DEG)
    def _(j):
        @pl.loop(0, K // LANES)
        def _(t):
            idx16 = idx_v[j, pl.ds(t * LANES, LANES)]
            cnt, last = plsc.scan_count(idx16)
            plsc.addupdate_scatter(
                deg_v, [idx16], cnt.astype(jnp.float32), mask=last
            )

    pltpu.sync_copy(deg_v.at[pl.ds(0, NPAD)], out_hbm.at[wid])


CHK = CH * K
KA = 64


@functools.partial(
    pl.kernel,
    out_type=jax.ShapeDtypeStruct((NC, HALF, DH), jnp.float32),
    mesh=_MESH,
    scratch_types=[
        pltpu.VMEM((CHK + LANES,), jnp.int32),
        pltpu.VMEM((CHK + LANES,), jnp.int32),
        pltpu.VMEM((4, KA, DH), jnp.float32),
        pltpu.VMEM_SHARED((RACC, DH), jnp.float32),
        pltpu.VMEM((LANES,), jnp.int32),
        pltpu.SemaphoreType.DMA,
        pltpu.SemaphoreType.DMA,
        pltpu.SemaphoreType.DMA,
        pltpu.SemaphoreType.DMA,
        pltpu.SemaphoreType.DMA,
        pltpu.SemaphoreType.DMA,
    ],
    compiler_params=_SC_PARAMS,
)
def _agg(hs_hbm, src_hbm, dst_hbm, meta_hbm, out_hbm,
         src_v, dst_v, rows_v, acc_sh, meta_v,
         gs0, gs1, gs2, gs3, ss0, ss1):
    c = lax.axis_index("c")
    s = lax.axis_index("s")
    wid = c * NS + s

    pltpu.sync_copy(meta_hbm, meta_v)
    meta = meta_v[...]
    nch = meta[0]
    base = pl.multiple_of(meta[1], HALF)

    pltpu.sync_copy(
        hs_hbm.at[c, pl.ds(base + s * RPT, RPT)],
        acc_sh.at[pl.ds(s * RPT, RPT)],
    )
    pltpu.sync_copy(src_hbm.at[wid], src_v.at[pl.ds(0, CHK)])
    pltpu.sync_copy(dst_hbm.at[wid], dst_v.at[pl.ds(0, CHK)])

    def _group(g, w):
        d = dst_v[pl.ds(g * LANES, LANES)]
        sv = src_v[pl.ds(g * LANES, LANES)]
        local = d - base
        ok = (local >= 0) & (local < HALF)
        plsc.store_compressed(src_v.at[pl.ds(w, LANES)], sv, mask=ok)
        plsc.store_compressed(dst_v.at[pl.ds(w, LANES)], local, mask=ok)
        return w + plsc.all_reduce_population_count(ok)[0]

    w = pl.loop(0, nch * (K // LANES), init_carry=jnp.int32(0))(_group)

    trash8 = HALF + (lax.iota(jnp.int32, LANES) & 7)
    zeros16 = jnp.zeros((LANES,), jnp.int32)
    target = ((w + KA - 1) // KA) * KA

    @pl.loop(0, KA // LANES)
    def _(t):
        pos = w + t * LANES

        @pl.when(pos < target)
        def _():
            src_v[pl.ds(pos, LANES)] = zeros16
            dst_v[pl.ds(pos, LANES)] = trash8

    nq = target // KA
    plsc.subcore_barrier()

    gsem = (gs0, gs1, gs2, gs3)
    ssem = (ss0, ss1)

    def _gather_desc(q, r):
        return pltpu.make_async_copy(
            hs_hbm.at[c].at[src_v.at[pl.ds(q * KA, KA)]], rows_v.at[r], gsem[r]
        )

    def _scatter_desc(q, r):
        return pltpu.make_async_copy(
            rows_v.at[r], acc_sh.at[dst_v.at[pl.ds(q * KA, KA)]], ssem[r % 2]
        )

    def _per_buf(q, fn):
        for r in range(4):
            @pl.when(q % 4 == r)
            def _(r=r):
                fn(q, r)

    @pl.when(nq > 0)
    def _():
        for i in range(2):
            @pl.when(i < nq)
            def _(i=i):
                _gather_desc(i, i).start()

        @pl.loop(0, nq)
        def _(q):
            @pl.when(q >= 2)
            def _():
                _per_buf(q - 2, lambda qq, r: _scatter_desc(qq, r).wait())

            @pl.when(q + 2 < nq)
            def _():
                _per_buf(q + 2, lambda qq, r: _gather_desc(qq, r).start())

            _per_buf(q, lambda qq, r: _gather_desc(qq, r).wait())
            _per_buf(q, lambda qq, r: _scatter_desc(qq, r).start(add=True))

        @pl.when(nq >= 2)
        def _():
            _per_buf(nq - 2, lambda qq, r: _scatter_desc(qq, r).wait())

        _per_buf(nq - 1, lambda qq, r: _scatter_desc(qq, r).wait())

    plsc.subcore_barrier()
    pltpu.sync_copy(
        acc_sh.at[pl.ds(s * RPT, RPT)],
        out_hbm.at[c, pl.ds(s * RPT, RPT)],
    )


_BM = 256
_MB = NPAD // _BM


def _tc1_body(degp_ref, x_ref, w_ref, hs_ref, dinv_ref):
    deg = jnp.sum(degp_ref[...], axis=0) + 1.0
    dinv = lax.rsqrt(deg)[:, None]
    x = x_ref[...]
    ha = jnp.dot(x, w_ref[0], preferred_element_type=jnp.float32)
    hb = jnp.dot(x, w_ref[1], preferred_element_type=jnp.float32)
    hs_ref[...] = jnp.concatenate(
        [(dinv * ha)[None], (dinv * hb)[None]], axis=0
    )
    dinv_ref[...] = dinv


def _tc1_call(degp, x_pad, w1):
    w1h = w1.reshape(F_IN, NC, DH).transpose(1, 0, 2)
    return pl.pallas_call(
        _tc1_body,
        grid=(_MB,),
        in_specs=[
            pl.BlockSpec((NC * NS, _BM), lambda i: (0, i)),
            pl.BlockSpec((_BM, F_IN), lambda i: (i, 0)),
            pl.BlockSpec((NC, F_IN, DH), lambda i: (0, 0, 0)),
        ],
        out_specs=[
            pl.BlockSpec((NC, _BM, DH), lambda i: (0, i, 0)),
            pl.BlockSpec((_BM, 1), lambda i: (i, 0)),
        ],
        out_shape=[
            jax.ShapeDtypeStruct((NC, NPAD, DH), jnp.float32),
            jax.ShapeDtypeStruct((NPAD, 1), jnp.float32),
        ],
    )(degp, x_pad, w1h)


def _tc2_body(acc_ref, dinv_ref, b1_ref, w2_ref, hs2_ref):
    dinv = dinv_ref[...]
    x2 = jnp.concatenate([acc_ref[0], acc_ref[1]], axis=1)
    x2 = jnp.maximum(dinv * x2 + b1_ref[...], 0.0)
    h2 = jnp.dot(x2, w2_ref[...], preferred_element_type=jnp.float32)
    hs2_ref[...] = jnp.broadcast_to((dinv * h2)[None], (NC, _BM, F_OUT))


def _tc2_call(acc1, dinv, b1, w2):
    return pl.pallas_call(
        _tc2_body,
        grid=(_MB,),
        in_specs=[
            pl.BlockSpec((NC, _BM, DH), lambda i: (0, i, 0)),
            pl.BlockSpec((_BM, 1), lambda i: (i, 0)),
            pl.BlockSpec((1, F_HID), lambda i: (0, 0)),
            pl.BlockSpec((F_HID, F_OUT), lambda i: (0, 0)),
        ],
        out_specs=pl.BlockSpec((NC, _BM, F_OUT), lambda i: (0, i, 0)),
        out_shape=jax.ShapeDtypeStruct((NC, NPAD, F_OUT), jnp.float32),
    )(acc1, dinv, b1, w2)


def _tc3_body(acc_ref, hs2_ref, dinv_ref, b2_ref, out_ref):
    y = acc_ref[0] + acc_ref[1] - hs2_ref[0]
    out_ref[...] = dinv_ref[...] * y + b2_ref[...]


def _tc3_call(acc2, hs2, dinv, b2):
    return pl.pallas_call(
        _tc3_body,
        grid=(_MB,),
        in_specs=[
            pl.BlockSpec((NC, _BM, F_OUT), lambda i: (0, i, 0)),
            pl.BlockSpec((1, _BM, F_OUT), lambda i: (0, i, 0)),
            pl.BlockSpec((_BM, 1), lambda i: (i, 0)),
            pl.BlockSpec((1, F_OUT), lambda i: (0, 0)),
        ],
        out_specs=pl.BlockSpec((_BM, F_OUT), lambda i: (i, 0)),
        out_shape=jax.ShapeDtypeStruct((NPAD, F_OUT), jnp.float32),
    )(acc2, hs2, dinv, b2)


def _agg_full(hs, src_t, dst_t, nch):
    halves = []
    for h in range(2):
        meta = jnp.array([nch, h * HALF] + [0] * (LANES - 2), jnp.int32)
        halves.append(_agg(hs, src_t, dst_t, meta))
    return jnp.concatenate(halves, axis=1)


def kernel(x, edge_index, W1, b1, W2, b2):
    src = edge_index[0].astype(jnp.int32)
    dst = edge_index[1].astype(jnp.int32)

    def _flat(v, nway, fill):
        per = E // nway
        body = v.reshape(nway, per)
        padb = jnp.full((nway, CHK - per), fill, jnp.int32)
        return jnp.concatenate([body, padb], axis=1)

    src16 = _flat(src, NS, 0)
    dst16 = _flat(dst, NS, TRASH)
    src_l1 = jnp.concatenate([src16, src16])
    dst_l1 = jnp.concatenate([dst16, dst16])
    src_l2 = _flat(src, NC * NS, 0)
    dst_l2 = _flat(dst, NC * NS, TRASH)
    dst32 = jnp.concatenate(
        [dst.reshape(NC * NS, E // (NC * NS)),
         jnp.full((NC * NS, CH_DEG * K - E // (NC * NS)), TRASH, jnp.int32)],
        axis=1,
    ).reshape(NC * NS, CH_DEG, K)
    x_pad = jnp.zeros((NPAD, F_IN), jnp.float32).at[:N].set(x)

    degp = _deg_kernel(dst32)
    hs1, dinv = _tc1_call(degp, x_pad, W1)
    acc1 = _agg_full(hs1, src_l1, dst_l1, CH)
    hs2 = _tc2_call(acc1, dinv, b1.reshape(1, F_HID), W2)
    acc2 = _agg_full(hs2, src_l2, dst_l2, CH_DEG)
    out = _tc3_call(acc2, hs2, dinv, b2.reshape(1, F_OUT))
    return out[:N]
